# baseline (device time: 20837 ns/iter reference)
import jax
import jax.numpy as jnp
from jax import lax
from jax.experimental import pallas as pl
from jax.experimental.pallas import tpu as pltpu

N_DEV = 4
B = 256
C = 1024


def kernel(x):
    m, n = x.shape
    nc = m // C
    nbc = C // B

    def body(x_hbm, out_hbm, xfull, obuf, comm_ref,
             rsems, wsems, send_sems, recv_sems):
        my = lax.axis_index("i")

        barrier_sem = pltpu.get_barrier_semaphore()
        for k in range(1, N_DEV):
            pl.semaphore_signal(
                barrier_sem, inc=1,
                device_id=(lax.rem(my + k, N_DEV),),
                device_id_type=pl.DeviceIdType.MESH,
            )

        def read(c):
            return pltpu.make_async_copy(
                x_hbm.at[pl.ds(c * C, C), :],
                xfull.at[pl.ds(c * C, C), :],
                rsems.at[c],
            )

        reads = [read(0)]
        reads[0].start()
        total = jnp.zeros((1, n), jnp.float32)
        for c in range(nc):
            if c + 1 < nc:
                nxt = read(c + 1)
                nxt.start()
                reads.append(nxt)
            reads[c].wait()
            total = total + jnp.sum(
                xfull[c * C:(c + 1) * C, :], axis=0, keepdims=True)
        comm_ref[0, :, :] = total

        pl.semaphore_wait(barrier_sem, N_DEV - 1)

        sends = []
        for k in range(1, N_DEV):
            rdma = pltpu.make_async_remote_copy(
                src_ref=comm_ref.at[0],
                dst_ref=comm_ref.at[N_DEV - k],
                send_sem=send_sems.at[k - 1],
                recv_sem=recv_sems.at[N_DEV - k],
                device_id=(lax.rem(my + k, N_DEV),),
                device_id_type=pl.DeviceIdType.MESH,
            )
            rdma.start()
            sends.append(rdma)

        r = lax.broadcasted_iota(jnp.int32, (B, B), 0)
        c_ = lax.broadcasted_iota(jnp.int32, (B, B), 1)
        L = (r >= c_).astype(jnp.bfloat16)

        for j in range(1, N_DEV):
            recv = pltpu.make_async_remote_copy(
                src_ref=comm_ref.at[0],
                dst_ref=comm_ref.at[j],
                send_sem=send_sems.at[0],
                recv_sem=recv_sems.at[j],
                device_id=(my,),
                device_id_type=pl.DeviceIdType.MESH,
            )
            recv.wait_recv()
        for rdma in sends:
            rdma.wait_send()

        tots = comm_ref[:, 0, :]
        j = lax.broadcasted_iota(jnp.int32, (N_DEV, n), 0)
        origin = lax.rem(my + j, N_DEV)
        offset = jnp.sum(jnp.where(origin < my, tots, 0.0), axis=0)

        carry = offset[None, :]
        wpend = [None, None]
        for c in range(nc):
            slot = c % 2
            if wpend[slot] is not None:
                wpend[slot].wait()
            for b in range(nbc):
                row0 = c * C + b * B
                xb = xfull[row0:row0 + B, :].astype(jnp.bfloat16)
                cs = jnp.dot(L, xb, preferred_element_type=jnp.float32)
                obuf[slot, b * B:(b + 1) * B, :] = cs + carry
                carry = carry + cs[B - 1:B, :]
            w = pltpu.make_async_copy(
                obuf.at[slot], out_hbm.at[pl.ds(c * C, C), :], wsems.at[slot])
            w.start()
            wpend[slot] = w
        wpend[0].wait()
        wpend[1].wait()

    return pl.pallas_call(
        body,
        out_shape=jax.ShapeDtypeStruct((m, n), x.dtype),
        in_specs=[pl.BlockSpec(memory_space=pl.ANY)],
        out_specs=pl.BlockSpec(memory_space=pl.ANY),
        scratch_shapes=[
            pltpu.VMEM((m, n), x.dtype),
            pltpu.VMEM((2, C, n), jnp.float32),
            pltpu.VMEM((N_DEV, 1, n), x.dtype),
            pltpu.SemaphoreType.DMA((m // C,)),
            pltpu.SemaphoreType.DMA((2,)),
            pltpu.SemaphoreType.DMA((N_DEV - 1,)),
            pltpu.SemaphoreType.DMA((N_DEV,)),
        ],
        compiler_params=pltpu.CompilerParams(collective_id=0),
    )(x)


# device time: 19789 ns/iter; 1.0530x vs baseline; 1.0530x over previous
import jax
import jax.numpy as jnp
from jax import lax
from jax.experimental import pallas as pl
from jax.experimental.pallas import tpu as pltpu

N_DEV = 4
B = 256
G = 2


def kernel(x):
    m, n = x.shape
    H = n // G

    def body(x_ref, out_ref, xb_ref, comm_ref, send_sems, recv_sems):
        my = lax.axis_index("i")

        barrier_sem = pltpu.get_barrier_semaphore()
        for k in range(1, N_DEV):
            pl.semaphore_signal(
                barrier_sem, inc=1,
                device_id=(lax.rem(my + k, N_DEV),),
                device_id_type=pl.DeviceIdType.MESH,
            )
        pl.semaphore_wait(barrier_sem, N_DEV - 1)

        ones8 = jnp.ones((8, m), jnp.bfloat16)

        sends = []
        for g in range(G):
            cols = slice(g * H, (g + 1) * H)
            xb_ref[:, cols] = x_ref[:, cols].astype(jnp.bfloat16)
            tg = jnp.dot(ones8, xb_ref[:, cols],
                         preferred_element_type=jnp.float32)
            comm_ref[0, :, cols] = tg[0:1, :]
            for k in range(1, N_DEV):
                rdma = pltpu.make_async_remote_copy(
                    src_ref=comm_ref.at[0, :, pl.ds(g * H, H)],
                    dst_ref=comm_ref.at[N_DEV - k, :, pl.ds(g * H, H)],
                    send_sem=send_sems.at[(k - 1) * G + g],
                    recv_sem=recv_sems.at[(N_DEV - k) * G + g],
                    device_id=(lax.rem(my + k, N_DEV),),
                    device_id_type=pl.DeviceIdType.MESH,
                )
                rdma.start()
                sends.append(rdma)

        r = lax.broadcasted_iota(jnp.int32, (B, B), 0)
        c_ = lax.broadcasted_iota(jnp.int32, (B, B), 1)
        L = (r >= c_).astype(jnp.bfloat16)

        for g in range(G):
            cols = slice(g * H, (g + 1) * H)
            for j in range(1, N_DEV):
                recv = pltpu.make_async_remote_copy(
                    src_ref=comm_ref.at[0, :, pl.ds(g * H, H)],
                    dst_ref=comm_ref.at[j, :, pl.ds(g * H, H)],
                    send_sem=send_sems.at[0],
                    recv_sem=recv_sems.at[j * G + g],
                    device_id=(my,),
                    device_id_type=pl.DeviceIdType.MESH,
                )
                recv.wait_recv()

            tots = comm_ref[:, 0, cols]
            jj = lax.broadcasted_iota(jnp.int32, (N_DEV, H), 0)
            origin = lax.rem(my + jj, N_DEV)
            offset = jnp.sum(jnp.where(origin < my, tots, 0.0), axis=0)

            carry = offset[None, :]
            for b in range(m // B):
                xb = xb_ref[b * B:(b + 1) * B, cols]
                cs = jnp.dot(L, xb, preferred_element_type=jnp.float32)
                out_ref[b * B:(b + 1) * B, cols] = cs + carry
                carry = carry + cs[B - 1:B, :]

        for rdma in sends:
            rdma.wait_send()

    return pl.pallas_call(
        body,
        out_shape=jax.ShapeDtypeStruct((m, n), x.dtype),
        in_specs=[pl.BlockSpec(memory_space=pltpu.VMEM)],
        out_specs=pl.BlockSpec(memory_space=pltpu.VMEM),
        scratch_shapes=[
            pltpu.VMEM((m, n), jnp.bfloat16),
            pltpu.VMEM((N_DEV, 1, n), x.dtype),
            pltpu.SemaphoreType.DMA(((N_DEV - 1) * G,)),
            pltpu.SemaphoreType.DMA((N_DEV * G,)),
        ],
        compiler_params=pltpu.CompilerParams(collective_id=0),
    )(x)


# device time: 19599 ns/iter; 1.0632x vs baseline; 1.0097x over previous
import jax
import jax.numpy as jnp
from jax import lax
from jax.experimental import pallas as pl
from jax.experimental.pallas import tpu as pltpu

N_DEV = 4
B = 256


def kernel(x):
    m, n = x.shape

    def body(x_ref, out_ref, comm_ref, send_sems, recv_sems):
        my = lax.axis_index("i")

        barrier_sem = pltpu.get_barrier_semaphore()
        for k in range(1, N_DEV):
            pl.semaphore_signal(
                barrier_sem, inc=1,
                device_id=(lax.rem(my + k, N_DEV),),
                device_id_type=pl.DeviceIdType.MESH,
            )
        pl.semaphore_wait(barrier_sem, N_DEV - 1)

        comm_ref[0, :, :] = jnp.sum(x_ref[:, :], axis=0, keepdims=True)

        sends = []
        for k in range(1, N_DEV):
            rdma = pltpu.make_async_remote_copy(
                src_ref=comm_ref.at[0],
                dst_ref=comm_ref.at[N_DEV - k],
                send_sem=send_sems.at[k - 1],
                recv_sem=recv_sems.at[N_DEV - k],
                device_id=(lax.rem(my + k, N_DEV),),
                device_id_type=pl.DeviceIdType.MESH,
            )
            rdma.start()
            sends.append(rdma)

        r = lax.broadcasted_iota(jnp.int32, (B, B), 0)
        c = lax.broadcasted_iota(jnp.int32, (B, B), 1)
        L = (r >= c).astype(jnp.bfloat16)

        for j in range(1, N_DEV):
            recv = pltpu.make_async_remote_copy(
                src_ref=comm_ref.at[0],
                dst_ref=comm_ref.at[j],
                send_sem=send_sems.at[0],
                recv_sem=recv_sems.at[j],
                device_id=(my,),
                device_id_type=pl.DeviceIdType.MESH,
            )
            recv.wait_recv()
        for rdma in sends:
            rdma.wait_send()

        tots = comm_ref[:, 0, :]
        j = lax.broadcasted_iota(jnp.int32, (N_DEV, n), 0)
        origin = lax.rem(my + j, N_DEV)
        offset = jnp.sum(jnp.where(origin < my, tots, 0.0), axis=0)

        carry = offset[None, :]
        for b in range(m // B):
            xb = x_ref[b * B:(b + 1) * B, :].astype(jnp.bfloat16)
            cs = jnp.dot(L, xb, preferred_element_type=jnp.float32)
            out_ref[b * B:(b + 1) * B, :] = cs + carry
            carry = carry + cs[B - 1:B, :]

    return pl.pallas_call(
        body,
        out_shape=jax.ShapeDtypeStruct((m, n), x.dtype),
        in_specs=[pl.BlockSpec(memory_space=pltpu.VMEM)],
        out_specs=pl.BlockSpec(memory_space=pltpu.VMEM),
        scratch_shapes=[
            pltpu.VMEM((N_DEV, 1, n), x.dtype),
            pltpu.SemaphoreType.DMA((N_DEV - 1,)),
            pltpu.SemaphoreType.DMA((N_DEV,)),
        ],
        compiler_params=pltpu.CompilerParams(collective_id=0),
    )(x)
